# baseline (speedup 1.0000x reference)
import jax
import jax.numpy as jnp
from jax.experimental import pallas as pl
from jax.experimental.pallas import tpu as pltpu

_F32 = jnp.float32


def _conv3(xb, wcat_ref, b_row):
    L, D = xb.shape
    z = jnp.zeros((1, D), xb.dtype)
    xpad = jnp.concatenate([z, xb, z], axis=0)
    ycat = jnp.dot(xpad, wcat_ref[...], preferred_element_type=_F32)
    return (ycat[0:L, 0:D] + ycat[1:L + 1, D:2 * D]
            + ycat[2:L + 2, 2 * D:3 * D] + b_row)


def _layernorm(h, g_row, b_row, ones_col):
    d = h.shape[1]
    s1 = jnp.dot(h, ones_col, preferred_element_type=_F32)
    s2 = jnp.dot(h * h, ones_col, preferred_element_type=_F32)
    m = s1 * (1.0 / d)
    v = s2 * (1.0 / d) - m * m
    rv = jax.lax.rsqrt(v + 1e-5)
    return (h - m) * rv * g_row + b_row


def _predictor(xb, w1_ref, w2_ref, v_ref, lw_col, ones_col):
    h = _conv3(xb, w1_ref, v_ref[0:1, :])
    h = jnp.maximum(h, 0.0)
    h = _layernorm(h, v_ref[1:2, :], v_ref[2:3, :], ones_col)
    h = _conv3(h, w2_ref, v_ref[3:4, :])
    h = jnp.maximum(h, 0.0)
    h = _layernorm(h, v_ref[4:5, :], v_ref[5:6, :], ones_col)
    out = jnp.dot(h, lw_col, preferred_element_type=_F32) + v_ref[7:8, 0:1]
    return out


def _body(x_ref, dur_ref, pt_ref, et_ref, bins2_ref, tri_ref, lw_ref,
          dw1, dw2, dv, pw1, pw2, pv, ew1, ew2, ev,
          ptab_ref, etab_ref, ml_ref,
          out_ref, ld_ref, pp_ref, ep_ref):
    S = x_ref.shape[1]
    T = pt_ref.shape[1]
    xb = x_ref[0]
    ones_col = jnp.full((xb.shape[1], 1), 1.0, _F32)

    ld_ref[0] = _predictor(xb, dw1, dw2, dv, lw_ref[:, 0:1], ones_col)

    dur_row = dur_ref[0]
    cum_f = jnp.dot(dur_row, tri_ref[...], preferred_element_type=_F32)
    cum_i = (cum_f + 0.5).astype(jnp.int32)
    dur_i = (dur_row + 0.5).astype(jnp.int32)
    cum_prev_i = cum_i - dur_i
    pos_i = jax.lax.broadcasted_iota(jnp.int32, (T, 1), 0)
    pos_i = jnp.where(pos_i < ml_ref[0:1, 0:1], pos_i, -1)
    onehot = (jnp.where(cum_prev_i <= pos_i, 1.0, 0.0)
              - jnp.where(cum_i <= pos_i, 1.0, 0.0))
    x_exp = jnp.dot(onehot, xb, preferred_element_type=_F32)

    pp_ref[0] = _predictor(x_exp, pw1, pw2, pv, lw_ref[:, 1:2], ones_col)
    ep_ref[0] = _predictor(x_exp, ew1, ew2, ev, lw_ref[:, 2:3], ones_col)

    lo = bins2_ref[0:1, :]
    hi = bins2_ref[1:2, :]
    vp = pt_ref[0]
    ve = et_ref[0]
    p_oh = jnp.where(lo < vp, 1.0, 0.0) - jnp.where(hi < vp, 1.0, 0.0)
    e_oh = jnp.where(lo < ve, 1.0, 0.0) - jnp.where(hi < ve, 1.0, 0.0)
    pemb = jnp.dot(p_oh, ptab_ref[...], preferred_element_type=_F32)
    eemb = jnp.dot(e_oh, etab_ref[...], preferred_element_type=_F32)

    out_ref[0] = x_exp + pemb + eemb


def _vecpack(p, D):
    return jnp.stack([
        p["c1b"], p["ln1g"], p["ln1b"],
        p["c2b"], p["ln2g"], p["ln2b"],
        p["lw"][:, 0], jnp.broadcast_to(p["lb"], (D,)),
    ], axis=0).astype(_F32)


def kernel(x, pitch_target, energy_target, params, src_mask, mel_mask,
           duration_target, max_len):
    B, S, D = x.shape
    T = mel_mask.shape[1]

    dur_f = duration_target.astype(_F32).reshape(B, 1, S)
    pt_col = pitch_target.reshape(B, T, 1)
    et_col = energy_target.reshape(B, T, 1)
    bins = jnp.linspace(0.0, 1.0, 255, dtype=_F32)
    bins_lo = jnp.concatenate([jnp.full((1,), -1e30, _F32), bins])
    bins_hi = jnp.concatenate([bins, jnp.full((1,), 1e30, _F32)])
    bins2 = jnp.stack([bins_lo, bins_hi], axis=0)
    ii = jax.lax.broadcasted_iota(jnp.int32, (S, S), 0)
    jj = jax.lax.broadcasted_iota(jnp.int32, (S, S), 1)
    tri = jnp.where(ii <= jj, 1.0, 0.0).astype(_F32)
    ml = jnp.broadcast_to(jnp.asarray(max_len, jnp.int32), (1, 1))

    dp, pp_, ep_ = params["dur"], params["pitch"], params["energy"]
    lw_cols = jnp.concatenate([
        dp["lw"], pp_["lw"], ep_["lw"],
        jnp.zeros((D, 5), _F32)], axis=1)
    wcat = lambda w: jnp.concatenate([w[0], w[1], w[2]], axis=1)
    operands = (
        x, dur_f, pt_col, et_col, bins2, tri, lw_cols,
        wcat(dp["c1w"]), wcat(dp["c2w"]), _vecpack(dp, D),
        wcat(pp_["c1w"]), wcat(pp_["c2w"]), _vecpack(pp_, D),
        wcat(ep_["c1w"]), wcat(ep_["c2w"]), _vecpack(ep_, D),
        params["pitch_table"], params["energy_table"], ml,
    )

    batch = lambda *blk: pl.BlockSpec(blk, lambda b: (b,) + (0,) * (len(blk) - 1))
    bcast = lambda *blk: pl.BlockSpec(blk, lambda b: (0,) * len(blk))
    wspecs = [bcast(D, 3 * D), bcast(D, 3 * D), bcast(8, D)]
    in_specs = [
        batch(1, S, D), batch(1, 1, S), batch(1, T, 1), batch(1, T, 1),
        bcast(2, 256), bcast(S, S), bcast(D, 8),
        *wspecs, *wspecs, *wspecs,
        bcast(256, D), bcast(256, D), bcast(1, 1),
    ]
    out_specs = [batch(1, T, D), batch(1, S, 1), batch(1, T, 1), batch(1, T, 1)]
    out_shape = [
        jax.ShapeDtypeStruct((B, T, D), _F32),
        jax.ShapeDtypeStruct((B, S, 1), _F32),
        jax.ShapeDtypeStruct((B, T, 1), _F32),
        jax.ShapeDtypeStruct((B, T, 1), _F32),
    ]

    out, ld, pp_col, ep_col = pl.pallas_call(
        _body,
        grid=(B,),
        in_specs=in_specs,
        out_specs=out_specs,
        out_shape=out_shape,
        compiler_params=pltpu.CompilerParams(
            dimension_semantics=("parallel",)),
    )(*operands)

    log_dur = jnp.where(src_mask, 0.0, ld.reshape(B, S))
    pitch_pred = jnp.where(mel_mask, 0.0, pp_col.reshape(B, T))
    energy_pred = jnp.where(mel_mask, 0.0, ep_col.reshape(B, T))
    return (out, log_dur, pitch_pred, energy_pred), (duration_target, mel_mask)


# speedup vs baseline: 1.0624x; 1.0624x over previous
import jax
import jax.numpy as jnp
from jax.experimental import pallas as pl
from jax.experimental.pallas import tpu as pltpu

_F32 = jnp.float32


def _shift_rows(a, k):
    L, D = a.shape
    z = jnp.zeros((abs(k), D), a.dtype)
    if k > 0:
        return jnp.concatenate([a[k:], z], axis=0)
    return jnp.concatenate([z, a[:k]], axis=0)


def _conv3(xb, w_ref, b_row):
    a0 = jnp.dot(xb, w_ref[0], preferred_element_type=_F32)
    a1 = jnp.dot(xb, w_ref[1], preferred_element_type=_F32)
    a2 = jnp.dot(xb, w_ref[2], preferred_element_type=_F32)
    return _shift_rows(a0, -1) + a1 + _shift_rows(a2, 1) + b_row


def _layernorm(h, g_row, b_row, ones_col):
    d = h.shape[1]
    s1 = jnp.dot(h, ones_col, preferred_element_type=_F32)
    s2 = jnp.dot(h * h, ones_col, preferred_element_type=_F32)
    m = s1 * (1.0 / d)
    v = s2 * (1.0 / d) - m * m
    rv = jax.lax.rsqrt(v + 1e-5)
    return (h - m) * rv * g_row + b_row


def _predictor(xb, w1_ref, w2_ref, v_ref, lw_col, ones_col):
    h = _conv3(xb, w1_ref, v_ref[0:1, :])
    h = jnp.maximum(h, 0.0)
    h = _layernorm(h, v_ref[1:2, :], v_ref[2:3, :], ones_col)
    h = _conv3(h, w2_ref, v_ref[3:4, :])
    h = jnp.maximum(h, 0.0)
    h = _layernorm(h, v_ref[4:5, :], v_ref[5:6, :], ones_col)
    out = jnp.dot(h, lw_col, preferred_element_type=_F32) + v_ref[7:8, 0:1]
    return out


def _body(x_ref, dur_ref, pt_ref, et_ref, bins2_ref, tri_ref, lw_ref,
          dw1, dw2, dv, pw1, pw2, pv, ew1, ew2, ev,
          ptab_ref, etab_ref, ml_ref,
          out_ref, ld_ref, pp_ref, ep_ref):
    S = x_ref.shape[1]
    T = pt_ref.shape[1]
    xb = x_ref[0]
    ones_col = jnp.full((xb.shape[1], 1), 1.0, _F32)

    ld_ref[0] = _predictor(xb, dw1, dw2, dv, lw_ref[:, 0:1], ones_col)

    dur_row = dur_ref[0]
    cum_f = jnp.dot(dur_row, tri_ref[...], preferred_element_type=_F32)
    cum_i = (cum_f + 0.5).astype(jnp.int32)
    dur_i = (dur_row + 0.5).astype(jnp.int32)
    cum_prev_i = cum_i - dur_i
    pos_i = jax.lax.broadcasted_iota(jnp.int32, (T, 1), 0)
    pos_i = jnp.where(pos_i < ml_ref[0:1, 0:1], pos_i, -1)
    onehot = (jnp.where(cum_prev_i <= pos_i, 1.0, 0.0)
              - jnp.where(cum_i <= pos_i, 1.0, 0.0))
    x_exp = jnp.dot(onehot, xb, preferred_element_type=_F32)

    pp_ref[0] = _predictor(x_exp, pw1, pw2, pv, lw_ref[:, 1:2], ones_col)
    ep_ref[0] = _predictor(x_exp, ew1, ew2, ev, lw_ref[:, 2:3], ones_col)

    lo = bins2_ref[0:1, :]
    hi = bins2_ref[1:2, :]
    vp = pt_ref[0]
    ve = et_ref[0]
    p_oh = jnp.where(lo < vp, 1.0, 0.0) - jnp.where(hi < vp, 1.0, 0.0)
    e_oh = jnp.where(lo < ve, 1.0, 0.0) - jnp.where(hi < ve, 1.0, 0.0)
    pemb = jnp.dot(p_oh, ptab_ref[...], preferred_element_type=_F32)
    eemb = jnp.dot(e_oh, etab_ref[...], preferred_element_type=_F32)

    out_ref[0] = x_exp + pemb + eemb


def _vecpack(p, D):
    return jnp.stack([
        p["c1b"], p["ln1g"], p["ln1b"],
        p["c2b"], p["ln2g"], p["ln2b"],
        p["lw"][:, 0], jnp.broadcast_to(p["lb"], (D,)),
    ], axis=0).astype(_F32)


def kernel(x, pitch_target, energy_target, params, src_mask, mel_mask,
           duration_target, max_len):
    B, S, D = x.shape
    T = mel_mask.shape[1]

    dur_f = duration_target.astype(_F32).reshape(B, 1, S)
    pt_col = pitch_target.reshape(B, T, 1)
    et_col = energy_target.reshape(B, T, 1)
    bins = jnp.linspace(0.0, 1.0, 255, dtype=_F32)
    bins_lo = jnp.concatenate([jnp.full((1,), -1e30, _F32), bins])
    bins_hi = jnp.concatenate([bins, jnp.full((1,), 1e30, _F32)])
    bins2 = jnp.stack([bins_lo, bins_hi], axis=0)
    ii = jax.lax.broadcasted_iota(jnp.int32, (S, S), 0)
    jj = jax.lax.broadcasted_iota(jnp.int32, (S, S), 1)
    tri = jnp.where(ii <= jj, 1.0, 0.0).astype(_F32)
    ml = jnp.broadcast_to(jnp.asarray(max_len, jnp.int32), (1, 1))

    dp, pp_, ep_ = params["dur"], params["pitch"], params["energy"]
    lw_cols = jnp.concatenate([
        dp["lw"], pp_["lw"], ep_["lw"],
        jnp.zeros((D, 5), _F32)], axis=1)
    operands = (
        x, dur_f, pt_col, et_col, bins2, tri, lw_cols,
        dp["c1w"], dp["c2w"], _vecpack(dp, D),
        pp_["c1w"], pp_["c2w"], _vecpack(pp_, D),
        ep_["c1w"], ep_["c2w"], _vecpack(ep_, D),
        params["pitch_table"], params["energy_table"], ml,
    )

    batch = lambda *blk: pl.BlockSpec(blk, lambda b: (b,) + (0,) * (len(blk) - 1))
    bcast = lambda *blk: pl.BlockSpec(blk, lambda b: (0,) * len(blk))
    wspecs = [bcast(3, D, D), bcast(3, D, D), bcast(8, D)]
    in_specs = [
        batch(1, S, D), batch(1, 1, S), batch(1, T, 1), batch(1, T, 1),
        bcast(2, 256), bcast(S, S), bcast(D, 8),
        *wspecs, *wspecs, *wspecs,
        bcast(256, D), bcast(256, D), bcast(1, 1),
    ]
    out_specs = [batch(1, T, D), batch(1, S, 1), batch(1, T, 1), batch(1, T, 1)]
    out_shape = [
        jax.ShapeDtypeStruct((B, T, D), _F32),
        jax.ShapeDtypeStruct((B, S, 1), _F32),
        jax.ShapeDtypeStruct((B, T, 1), _F32),
        jax.ShapeDtypeStruct((B, T, 1), _F32),
    ]

    out, ld, pp_col, ep_col = pl.pallas_call(
        _body,
        grid=(B,),
        in_specs=in_specs,
        out_specs=out_specs,
        out_shape=out_shape,
        compiler_params=pltpu.CompilerParams(
            dimension_semantics=("parallel",)),
    )(*operands)

    log_dur = jnp.where(src_mask, 0.0, ld.reshape(B, S))
    pitch_pred = jnp.where(mel_mask, 0.0, pp_col.reshape(B, T))
    energy_pred = jnp.where(mel_mask, 0.0, ep_col.reshape(B, T))
    return (out, log_dur, pitch_pred, energy_pred), (duration_target, mel_mask)
